# baseline (speedup 1.0000x reference)
import functools

import jax
import jax.numpy as jnp
from jax import lax
from jax.experimental import pallas as pl
from jax.experimental.pallas import tpu as pltpu
from jax.experimental.pallas import tpu_sc as plsc

B = 16384
D = 64
NEG = 20
NC = 2
NS = 16
NW = NC * NS
BPW = B // NW
CHUNK = 64
NCHUNK = BPW // CHUNK
IDX_DMA = 128
SCORE_W = 32


CONV_W = 256
N_FULL = (1000000 - 64) // CONV_W
CONV_PAIRS = N_FULL // (2 * NW)
N_EVEN = CONV_PAIRS * 2 * NW


@functools.partial(
    pl.kernel,
    out_type=(
        jax.ShapeDtypeStruct((1000000 * D,), jnp.float32),
        jax.ShapeDtypeStruct((1000000 * D,), jnp.float32),
    ),
    mesh=plsc.VectorSubcoreMesh(
        core_axis_name="c", subcore_axis_name="s", num_cores=NC, num_subcores=NS
    ),
    scratch_types=[
        pltpu.VMEM((D, CONV_W + 1), jnp.float32),
        pltpu.VMEM((D, CONV_W + 1), jnp.float32),
        pltpu.VMEM((CONV_W * D,), jnp.float32),
        pltpu.VMEM((CONV_W * D,), jnp.float32),
        pltpu.SemaphoreType.DMA,
        pltpu.SemaphoreType.DMA,
        pltpu.SemaphoreType.DMA,
        pltpu.SemaphoreType.DMA,
    ],
    compiler_params=pltpu.CompilerParams(
        needs_layout_passes=False, use_tc_tiling_on_sc=True
    ),
)
def _sc_convert(vT, uT, v_tail, u_tail, v_lin, u_lin,
                inb0, inb1, outb0, outb1, si0, si1, so0, so1):
    wid = lax.axis_index("s") * NC + lax.axis_index("c")
    lanes = lax.iota(jnp.int32, 16)
    inb = (inb0, inb1)
    outb = (outb0, outb1)
    sin = (si0, si1)
    sout = (so0, so1)

    def transpose_block(src, dst):
        def rowpair(r, carry):
            for s in range(2):
                i = r * 2 + s
                ci = jnp.broadcast_to(i, (16,))
                for j in range(4):
                    g = plsc.load_gather(src, [lanes + 16 * j, ci])
                    dst[pl.ds(i * D + 16 * j, 16)] = g
            return carry

        lax.fori_loop(0, CONV_W // 2, rowpair, 0)

    for tab, tail, lin in ((vT, v_tail, v_lin), (uT, u_tail, u_lin)):
        def col0(g):
            return pl.multiple_of((wid * 2 * CONV_PAIRS + g) * CONV_W, 128)

        pltpu.async_copy(tab.at[:, pl.ds(col0(0), CONV_W)],
                         inb[0].at[:, pl.ds(0, CONV_W)], sin[0])

        def pair(it2, carry):
            for phase in range(2):
                g = 2 * it2 + phase
                buf = phase
                @pl.when(g < 2 * CONV_PAIRS - 1)
                def _():
                    pltpu.async_copy(tab.at[:, pl.ds(col0(g + 1), CONV_W)],
                                     inb[1 - buf].at[:, pl.ds(0, CONV_W)],
                                     sin[1 - buf])

                pltpu.make_async_copy(tab.at[:, pl.ds(0, CONV_W)],
                                      inb[buf].at[:, pl.ds(0, CONV_W)],
                                      sin[buf]).wait()

                @pl.when(it2 >= 1)
                def _():
                    pltpu.make_async_copy(
                        outb[buf], lin.at[pl.ds(0, CONV_W * D)],
                        sout[buf]).wait()

                transpose_block(inb[buf], outb[buf])
                pltpu.async_copy(outb[buf],
                                 lin.at[pl.ds(col0(g) * D, CONV_W * D)],
                                 sout[buf])
            return carry

        lax.fori_loop(0, CONV_PAIRS, pair, 0)
        for buf in range(2):
            pltpu.make_async_copy(outb[buf], lin.at[pl.ds(0, CONV_W * D)],
                                  sout[buf]).wait()

        @pl.when(wid < 2)
        def _extra():
            c0 = pl.multiple_of((N_EVEN + wid) * CONV_W, 128)
            pltpu.async_copy(tab.at[:, pl.ds(c0, CONV_W)],
                             inb[0].at[:, pl.ds(0, CONV_W)], sin[0]).wait()
            transpose_block(inb[0], outb[0])
            pltpu.async_copy(outb[0], lin.at[pl.ds(c0 * D, CONV_W * D)],
                             sout[0]).wait()

        @pl.when(wid == 2)
        def _tail():
            c0 = N_FULL * CONV_W
            pltpu.sync_copy(tail, outb[0].at[pl.ds(0, 64 * D)])
            pltpu.sync_copy(outb[0].at[pl.ds(0, 64 * D)],
                            lin.at[pl.ds(c0 * D, 64 * D)])


def _hsum(x, perms):
    for p in perms:
        x = x + x.at[p].get(mode="promise_in_bounds")
    return x


@functools.partial(
    pl.kernel,
    out_type=jax.ShapeDtypeStruct((B * SCORE_W,), jnp.float32),
    mesh=plsc.VectorSubcoreMesh(
        core_axis_name="c", subcore_axis_name="s", num_cores=NC, num_subcores=NS
    ),
    scratch_types=[
        pltpu.VMEM((BPW,), jnp.int32),
        pltpu.VMEM((BPW,), jnp.int32),
        pltpu.VMEM((BPW * NEG,), jnp.int32),
        pltpu.VMEM((CHUNK, D), jnp.float32),
        pltpu.VMEM((CHUNK, D), jnp.float32),
        pltpu.VMEM((CHUNK * NEG, D), jnp.float32),
        pltpu.VMEM((CHUNK * SCORE_W,), jnp.float32),
        pltpu.SemaphoreType.DMA,
    ],
    compiler_params=pltpu.CompilerParams(
        needs_layout_passes=False, use_tc_tiling_on_sc=False
    ),
)
def _sc_scores(v_emb, u_emb, v_h, up_h, un_h, out,
               v_idx, up_idx, un_idx, v_rows, up_rows, un_rows, scores, sem):
    wid = lax.axis_index("s") * NC + lax.axis_index("c")
    base = wid * BPW

    pltpu.sync_copy(v_h.at[pl.ds(base, BPW)], v_idx)
    pltpu.sync_copy(up_h.at[pl.ds(base, BPW)], up_idx)
    pltpu.sync_copy(un_h.at[pl.ds(base * NEG, BPW * NEG)], un_idx)

    lanes = lax.iota(jnp.int32, 16)
    perms = [lanes ^ k for k in (8, 4, 2, 1)]
    lane0 = lanes == 0
    zeros16 = jnp.zeros((16,), jnp.float32)

    for c in range(NCHUNK):
        cb = c * CHUNK
        cps = [
            pltpu.async_copy(v_emb.at[v_idx.at[pl.ds(cb, CHUNK)]], v_rows, sem),
            pltpu.async_copy(u_emb.at[up_idx.at[pl.ds(cb, CHUNK)]], up_rows, sem),
        ]
        for k in range(CHUNK * NEG // IDX_DMA):
            cps.append(pltpu.async_copy(
                u_emb.at[un_idx.at[pl.ds(cb * NEG + k * IDX_DMA, IDX_DMA)]],
                un_rows.at[pl.ds(k * IDX_DMA, IDX_DMA)], sem))

        def zpad(e, carry):
            scores[pl.ds(e * SCORE_W + 16, 16)] = zeros16
            return carry

        lax.fori_loop(0, CHUNK, zpad, 0)
        for cp in cps:
            cp.wait()

        def elem(e, carry):
            obase = e * SCORE_W
            vr = [v_rows[e, pl.ds(16 * j, 16)] for j in range(4)]
            up = [up_rows[e, pl.ds(16 * j, 16)] for j in range(4)]
            acc = vr[0] * up[0]
            for j in range(1, 4):
                acc = acc + vr[j] * up[j]
            s = _hsum(acc, perms)
            plsc.store_scatter(scores, [jnp.broadcast_to(obase, (16,))], s,
                               mask=lane0)
            nv = [-x for x in vr]
            for n in range(NEG):
                r = e * NEG + n
                un = [un_rows[r, pl.ds(16 * j, 16)] for j in range(4)]
                a = nv[0] * un[0]
                for j in range(1, 4):
                    a = a + nv[j] * un[j]
                s = _hsum(a, perms)
                plsc.store_scatter(scores,
                                   [jnp.broadcast_to(obase + n + 1, (16,))], s,
                                   mask=lane0)
            return carry

        lax.fori_loop(0, CHUNK, elem, 0)
        pltpu.sync_copy(scores,
                        out.at[pl.ds((base + cb) * SCORE_W, CHUNK * SCORE_W)])


def _tc_reduce(scores):
    nblk = 16

    def body(s_ref, o_ref):
        i = pl.program_id(0)

        @pl.when(i == 0)
        def _init():
            o_ref[0, 0] = jnp.float32(0.0)

        x = s_ref[...]
        col = lax.broadcasted_iota(jnp.int32, x.shape, 1)
        ls = jnp.minimum(x, 0.0) - jnp.log1p(jnp.exp(-jnp.abs(x)))
        ls = jnp.where(col < NEG + 1, ls, 0.0)
        o_ref[0, 0] += jnp.sum(ls)

    return pl.pallas_call(
        body,
        grid=(nblk,),
        in_specs=[pl.BlockSpec((B // nblk, SCORE_W), lambda i: (i, 0))],
        out_specs=pl.BlockSpec(memory_space=pltpu.SMEM),
        out_shape=jax.ShapeDtypeStruct((1, 1), jnp.float32),
    )(scores)


def kernel(v_emb, u_emb, v, u_pos, u_neg):
    v = v.astype(jnp.int32)
    u_pos = u_pos.astype(jnp.int32)
    u_neg_flat = u_neg.astype(jnp.int32).reshape(B * NEG)
    v_tail = lax.slice(v_emb, (N_FULL * CONV_W, 0), (1000000, D)).reshape(-1)
    u_tail = lax.slice(u_emb, (N_FULL * CONV_W, 0), (1000000, D)).reshape(-1)
    v_lin, u_lin = _sc_convert(v_emb.T, u_emb.T, v_tail, u_tail)
    scores = _sc_scores(v_lin.reshape(1000000, D), u_lin.reshape(1000000, D),
                        v, u_pos, u_neg_flat)
    tot = _tc_reduce(scores.reshape(B, SCORE_W))
    return -tot[0, 0]


# speedup vs baseline: 1.4890x; 1.4890x over previous
import functools

import jax
import jax.numpy as jnp
from jax import lax
from jax.experimental import pallas as pl
from jax.experimental.pallas import tpu as pltpu
from jax.experimental.pallas import tpu_sc as plsc

B = 16384
D = 64
NEG = 20
NC = 2
NS = 16
NW = NC * NS
BPW = B // NW
CHUNK = 64
NCHUNK = BPW // CHUNK
IDX_DMA = 128
SCORE_W = 32


CONV_W = 256
N_FULL = (1000000 - 64) // CONV_W
CONV_PAIRS = N_FULL // (2 * NW)
N_EVEN = CONV_PAIRS * 2 * NW


@functools.partial(
    pl.kernel,
    out_type=(
        jax.ShapeDtypeStruct((1000000 * D,), jnp.float32),
        jax.ShapeDtypeStruct((1000000 * D,), jnp.float32),
    ),
    mesh=plsc.VectorSubcoreMesh(
        core_axis_name="c", subcore_axis_name="s", num_cores=NC, num_subcores=NS
    ),
    scratch_types=[
        pltpu.VMEM((D, CONV_W), jnp.float32),
        pltpu.VMEM((D, CONV_W), jnp.float32),
        pltpu.VMEM((D * (CONV_W + 1),), jnp.float32),
        pltpu.VMEM((CONV_W * D,), jnp.float32),
        pltpu.VMEM((CONV_W * D,), jnp.float32),
        pltpu.SemaphoreType.DMA,
        pltpu.SemaphoreType.DMA,
        pltpu.SemaphoreType.DMA,
        pltpu.SemaphoreType.DMA,
    ],
    compiler_params=pltpu.CompilerParams(
        needs_layout_passes=False, use_tc_tiling_on_sc=True
    ),
)
def _sc_convert(vT, uT, v_tail, u_tail, v_lin, u_lin,
                inb0, inb1, pad, outb0, outb1, si0, si1, so0, so1):
    wid = lax.axis_index("s") * NC + lax.axis_index("c")
    lanes = lax.iota(jnp.int32, 16)
    inb = (inb0, inb1)
    outb = (outb0, outb1)
    sin = (si0, si1)
    sout = (so0, so1)

    STRIDE = CONV_W + 1
    rowsel = lanes * STRIDE

    def transpose_block(src, dst):
        def repack(d, carry):
            for j in range(CONV_W // 16):
                pad[pl.ds(d * STRIDE + 16 * j, 16)] = src[d, pl.ds(16 * j, 16)]
            return carry

        lax.fori_loop(0, D, repack, 0)

        def rowpair(r, carry):
            for s in range(2):
                i = r * 2 + s
                ci = jnp.broadcast_to(i, (16,))
                for j in range(4):
                    g = plsc.load_gather(pad, [rowsel + (16 * j * STRIDE + i)])
                    dst[pl.ds(i * D + 16 * j, 16)] = g
            return carry

        lax.fori_loop(0, CONV_W // 2, rowpair, 0)

    for tab, tail, lin in ((vT, v_tail, v_lin), (uT, u_tail, u_lin)):
        def col0(g):
            return pl.multiple_of((wid * 2 * CONV_PAIRS + g) * CONV_W, 128)

        pltpu.async_copy(tab.at[:, pl.ds(col0(0), CONV_W)],
                         inb[0].at[:, pl.ds(0, CONV_W)], sin[0])

        def pair(it2, carry):
            for phase in range(2):
                g = 2 * it2 + phase
                buf = phase
                @pl.when(g < 2 * CONV_PAIRS - 1)
                def _():
                    pltpu.async_copy(tab.at[:, pl.ds(col0(g + 1), CONV_W)],
                                     inb[1 - buf].at[:, pl.ds(0, CONV_W)],
                                     sin[1 - buf])

                pltpu.make_async_copy(tab.at[:, pl.ds(0, CONV_W)],
                                      inb[buf].at[:, pl.ds(0, CONV_W)],
                                      sin[buf]).wait()

                @pl.when(it2 >= 1)
                def _():
                    pltpu.make_async_copy(
                        outb[buf], lin.at[pl.ds(0, CONV_W * D)],
                        sout[buf]).wait()

                transpose_block(inb[buf], outb[buf])
                pltpu.async_copy(outb[buf],
                                 lin.at[pl.ds(col0(g) * D, CONV_W * D)],
                                 sout[buf])
            return carry

        lax.fori_loop(0, CONV_PAIRS, pair, 0)
        for buf in range(2):
            pltpu.make_async_copy(outb[buf], lin.at[pl.ds(0, CONV_W * D)],
                                  sout[buf]).wait()

        @pl.when(wid < 2)
        def _extra():
            c0 = pl.multiple_of((N_EVEN + wid) * CONV_W, 128)
            pltpu.async_copy(tab.at[:, pl.ds(c0, CONV_W)],
                             inb[0].at[:, pl.ds(0, CONV_W)], sin[0]).wait()
            transpose_block(inb[0], outb[0])
            pltpu.async_copy(outb[0], lin.at[pl.ds(c0 * D, CONV_W * D)],
                             sout[0]).wait()

        @pl.when(wid == 2)
        def _tail():
            c0 = N_FULL * CONV_W
            pltpu.sync_copy(tail, outb[0].at[pl.ds(0, 64 * D)])
            pltpu.sync_copy(outb[0].at[pl.ds(0, 64 * D)],
                            lin.at[pl.ds(c0 * D, 64 * D)])


def _hsum(x, perms):
    for p in perms:
        x = x + x.at[p].get(mode="promise_in_bounds")
    return x


@functools.partial(
    pl.kernel,
    out_type=jax.ShapeDtypeStruct((B * SCORE_W,), jnp.float32),
    mesh=plsc.VectorSubcoreMesh(
        core_axis_name="c", subcore_axis_name="s", num_cores=NC, num_subcores=NS
    ),
    scratch_types=[
        pltpu.VMEM((BPW,), jnp.int32),
        pltpu.VMEM((BPW,), jnp.int32),
        pltpu.VMEM((BPW * NEG,), jnp.int32),
        pltpu.VMEM((CHUNK, D), jnp.float32),
        pltpu.VMEM((CHUNK, D), jnp.float32),
        pltpu.VMEM((CHUNK * NEG, D), jnp.float32),
        pltpu.VMEM((CHUNK * SCORE_W,), jnp.float32),
        pltpu.SemaphoreType.DMA,
    ],
    compiler_params=pltpu.CompilerParams(
        needs_layout_passes=False, use_tc_tiling_on_sc=False
    ),
)
def _sc_scores(v_emb, u_emb, v_h, up_h, un_h, out,
               v_idx, up_idx, un_idx, v_rows, up_rows, un_rows, scores, sem):
    wid = lax.axis_index("s") * NC + lax.axis_index("c")
    base = wid * BPW

    pltpu.sync_copy(v_h.at[pl.ds(base, BPW)], v_idx)
    pltpu.sync_copy(up_h.at[pl.ds(base, BPW)], up_idx)
    pltpu.sync_copy(un_h.at[pl.ds(base * NEG, BPW * NEG)], un_idx)

    lanes = lax.iota(jnp.int32, 16)
    perms = [lanes ^ k for k in (8, 4, 2, 1)]
    lane0 = lanes == 0
    zeros16 = jnp.zeros((16,), jnp.float32)

    for c in range(NCHUNK):
        cb = c * CHUNK
        cps = [
            pltpu.async_copy(v_emb.at[v_idx.at[pl.ds(cb, CHUNK)]], v_rows, sem),
            pltpu.async_copy(u_emb.at[up_idx.at[pl.ds(cb, CHUNK)]], up_rows, sem),
        ]
        for k in range(CHUNK * NEG // IDX_DMA):
            cps.append(pltpu.async_copy(
                u_emb.at[un_idx.at[pl.ds(cb * NEG + k * IDX_DMA, IDX_DMA)]],
                un_rows.at[pl.ds(k * IDX_DMA, IDX_DMA)], sem))

        def zpad(e, carry):
            scores[pl.ds(e * SCORE_W + 16, 16)] = zeros16
            return carry

        lax.fori_loop(0, CHUNK, zpad, 0)
        for cp in cps:
            cp.wait()

        def elem(e, carry):
            obase = e * SCORE_W
            vr = [v_rows[e, pl.ds(16 * j, 16)] for j in range(4)]
            up = [up_rows[e, pl.ds(16 * j, 16)] for j in range(4)]
            acc = vr[0] * up[0]
            for j in range(1, 4):
                acc = acc + vr[j] * up[j]
            s = _hsum(acc, perms)
            plsc.store_scatter(scores, [jnp.broadcast_to(obase, (16,))], s,
                               mask=lane0)
            nv = [-x for x in vr]
            for n in range(NEG):
                r = e * NEG + n
                un = [un_rows[r, pl.ds(16 * j, 16)] for j in range(4)]
                a = nv[0] * un[0]
                for j in range(1, 4):
                    a = a + nv[j] * un[j]
                s = _hsum(a, perms)
                plsc.store_scatter(scores,
                                   [jnp.broadcast_to(obase + n + 1, (16,))], s,
                                   mask=lane0)
            return carry

        lax.fori_loop(0, CHUNK, elem, 0)
        pltpu.sync_copy(scores,
                        out.at[pl.ds((base + cb) * SCORE_W, CHUNK * SCORE_W)])


def _tc_reduce(scores):
    nblk = 16

    def body(s_ref, o_ref):
        i = pl.program_id(0)

        @pl.when(i == 0)
        def _init():
            o_ref[0, 0] = jnp.float32(0.0)

        x = s_ref[...]
        col = lax.broadcasted_iota(jnp.int32, x.shape, 1)
        ls = jnp.minimum(x, 0.0) - jnp.log1p(jnp.exp(-jnp.abs(x)))
        ls = jnp.where(col < NEG + 1, ls, 0.0)
        o_ref[0, 0] += jnp.sum(ls)

    return pl.pallas_call(
        body,
        grid=(nblk,),
        in_specs=[pl.BlockSpec((B // nblk, SCORE_W), lambda i: (i, 0))],
        out_specs=pl.BlockSpec(memory_space=pltpu.SMEM),
        out_shape=jax.ShapeDtypeStruct((1, 1), jnp.float32),
    )(scores)


def kernel(v_emb, u_emb, v, u_pos, u_neg):
    v = v.astype(jnp.int32)
    u_pos = u_pos.astype(jnp.int32)
    u_neg_flat = u_neg.astype(jnp.int32).reshape(B * NEG)
    v_tail = lax.slice(v_emb, (N_FULL * CONV_W, 0), (1000000, D)).reshape(-1)
    u_tail = lax.slice(u_emb, (N_FULL * CONV_W, 0), (1000000, D)).reshape(-1)
    v_lin, u_lin = _sc_convert(v_emb.T, u_emb.T, v_tail, u_tail)
    scores = _sc_scores(v_lin.reshape(1000000, D), u_lin.reshape(1000000, D),
                        v, u_pos, u_neg_flat)
    tot = _tc_reduce(scores.reshape(B, SCORE_W))
    return -tot[0, 0]


# speedup vs baseline: 4.0119x; 2.6943x over previous
import functools

import jax
import jax.numpy as jnp
from jax import lax
from jax.experimental import pallas as pl
from jax.experimental.pallas import tpu as pltpu
from jax.experimental.pallas import tpu_sc as plsc

B = 16384
D = 64
NEG = 20
NC = 2
NS = 16
NW = NC * NS
BPW = B // NW
CHUNK = 64
NCHUNK = BPW // CHUNK
IDX_DMA = 128
SCORE_W = 32


CONV_W = 256
N_FULL = (1000000 - 64) // CONV_W
CONV_PAIRS = N_FULL // (2 * NW)
N_EVEN = CONV_PAIRS * 2 * NW


@functools.partial(
    pl.kernel,
    out_type=(
        jax.ShapeDtypeStruct((1000000 * D,), jnp.float32),
        jax.ShapeDtypeStruct((1000000 * D,), jnp.float32),
    ),
    mesh=plsc.VectorSubcoreMesh(
        core_axis_name="c", subcore_axis_name="s", num_cores=NC, num_subcores=NS
    ),
    scratch_types=[
        pltpu.VMEM((D, CONV_W), jnp.float32),
        pltpu.VMEM((D, CONV_W), jnp.float32),
        pltpu.VMEM((CONV_W * D,), jnp.float32),
        pltpu.VMEM((CONV_W * D,), jnp.float32),
        pltpu.SemaphoreType.DMA,
        pltpu.SemaphoreType.DMA,
        pltpu.SemaphoreType.DMA,
        pltpu.SemaphoreType.DMA,
    ],
    compiler_params=pltpu.CompilerParams(
        needs_layout_passes=False, use_tc_tiling_on_sc=True
    ),
)
def _sc_convert(vT, uT, v_tail, u_tail, v_lin, u_lin,
                inb0, inb1, outb0, outb1, si0, si1, so0, so1):
    wid = lax.axis_index("s") * NC + lax.axis_index("c")
    lanes = lax.iota(jnp.int32, 16)
    inb = (inb0, inb1)
    outb = (outb0, outb1)
    sin = (si0, si1)
    sout = (so0, so1)

    perms = [lanes ^ s for s in (1, 2, 4, 8)]
    lmasks = [(lanes & s) == 0 for s in (1, 2, 4, 8)]

    def transpose_block(src, dst):
        def subtile(t, carry):
            r0 = (t % 4) * 16
            c0 = (t // 4) * 16
            x = [src[r0 + r, pl.ds(c0, 16)] for r in range(16)]
            for k, s in enumerate((1, 2, 4, 8)):
                perm, m = perms[k], lmasks[k]
                y = list(x)
                for r in range(0, 16):
                    if r & s:
                        continue
                    rp = r | s
                    ga = x[rp].at[perm].get(mode="promise_in_bounds")
                    gb = x[r].at[perm].get(mode="promise_in_bounds")
                    y[r] = jnp.where(m, x[r], ga)
                    y[rp] = jnp.where(m, gb, x[rp])
                x = y
            for i in range(16):
                dst[pl.ds((c0 + i) * D + r0, 16)] = x[i]
            return carry

        lax.fori_loop(0, (D // 16) * (CONV_W // 16), subtile, 0)

    for tab, tail, lin in ((vT, v_tail, v_lin), (uT, u_tail, u_lin)):
        def col0(g):
            return pl.multiple_of((wid * 2 * CONV_PAIRS + g) * CONV_W, 128)

        pltpu.async_copy(tab.at[:, pl.ds(col0(0), CONV_W)],
                         inb[0].at[:, pl.ds(0, CONV_W)], sin[0])

        def pair(it2, carry):
            for phase in range(2):
                g = 2 * it2 + phase
                buf = phase
                @pl.when(g < 2 * CONV_PAIRS - 1)
                def _():
                    pltpu.async_copy(tab.at[:, pl.ds(col0(g + 1), CONV_W)],
                                     inb[1 - buf].at[:, pl.ds(0, CONV_W)],
                                     sin[1 - buf])

                pltpu.make_async_copy(tab.at[:, pl.ds(0, CONV_W)],
                                      inb[buf].at[:, pl.ds(0, CONV_W)],
                                      sin[buf]).wait()

                @pl.when(it2 >= 1)
                def _():
                    pltpu.make_async_copy(
                        outb[buf], lin.at[pl.ds(0, CONV_W * D)],
                        sout[buf]).wait()

                transpose_block(inb[buf], outb[buf])
                pltpu.async_copy(outb[buf],
                                 lin.at[pl.ds(col0(g) * D, CONV_W * D)],
                                 sout[buf])
            return carry

        lax.fori_loop(0, CONV_PAIRS, pair, 0)
        for buf in range(2):
            pltpu.make_async_copy(outb[buf], lin.at[pl.ds(0, CONV_W * D)],
                                  sout[buf]).wait()

        @pl.when(wid < 2)
        def _extra():
            c0 = pl.multiple_of((N_EVEN + wid) * CONV_W, 128)
            pltpu.async_copy(tab.at[:, pl.ds(c0, CONV_W)],
                             inb[0].at[:, pl.ds(0, CONV_W)], sin[0]).wait()
            transpose_block(inb[0], outb[0])
            pltpu.async_copy(outb[0], lin.at[pl.ds(c0 * D, CONV_W * D)],
                             sout[0]).wait()

        @pl.when(wid == 2)
        def _tail():
            c0 = N_FULL * CONV_W
            pltpu.sync_copy(tail, outb[0].at[pl.ds(0, 64 * D)])
            pltpu.sync_copy(outb[0].at[pl.ds(0, 64 * D)],
                            lin.at[pl.ds(c0 * D, 64 * D)])


def _hsum(x, perms):
    for p in perms:
        x = x + x.at[p].get(mode="promise_in_bounds")
    return x


@functools.partial(
    pl.kernel,
    out_type=jax.ShapeDtypeStruct((B * SCORE_W,), jnp.float32),
    mesh=plsc.VectorSubcoreMesh(
        core_axis_name="c", subcore_axis_name="s", num_cores=NC, num_subcores=NS
    ),
    scratch_types=[
        pltpu.VMEM((BPW,), jnp.int32),
        pltpu.VMEM((BPW,), jnp.int32),
        pltpu.VMEM((BPW * NEG,), jnp.int32),
        pltpu.VMEM((CHUNK, D), jnp.float32),
        pltpu.VMEM((CHUNK, D), jnp.float32),
        pltpu.VMEM((CHUNK * NEG, D), jnp.float32),
        pltpu.VMEM((CHUNK * SCORE_W,), jnp.float32),
        pltpu.SemaphoreType.DMA,
    ],
    compiler_params=pltpu.CompilerParams(
        needs_layout_passes=False, use_tc_tiling_on_sc=False
    ),
)
def _sc_scores(v_emb, u_emb, v_h, up_h, un_h, out,
               v_idx, up_idx, un_idx, v_rows, up_rows, un_rows, scores, sem):
    wid = lax.axis_index("s") * NC + lax.axis_index("c")
    base = wid * BPW

    pltpu.sync_copy(v_h.at[pl.ds(base, BPW)], v_idx)
    pltpu.sync_copy(up_h.at[pl.ds(base, BPW)], up_idx)
    pltpu.sync_copy(un_h.at[pl.ds(base * NEG, BPW * NEG)], un_idx)

    lanes = lax.iota(jnp.int32, 16)
    perms = [lanes ^ k for k in (8, 4, 2, 1)]
    lane0 = lanes == 0
    zeros16 = jnp.zeros((16,), jnp.float32)

    for c in range(NCHUNK):
        cb = c * CHUNK
        cps = [
            pltpu.async_copy(v_emb.at[v_idx.at[pl.ds(cb, CHUNK)]], v_rows, sem),
            pltpu.async_copy(u_emb.at[up_idx.at[pl.ds(cb, CHUNK)]], up_rows, sem),
        ]
        for k in range(CHUNK * NEG // IDX_DMA):
            cps.append(pltpu.async_copy(
                u_emb.at[un_idx.at[pl.ds(cb * NEG + k * IDX_DMA, IDX_DMA)]],
                un_rows.at[pl.ds(k * IDX_DMA, IDX_DMA)], sem))

        def zpad(e, carry):
            scores[pl.ds(e * SCORE_W + 16, 16)] = zeros16
            return carry

        lax.fori_loop(0, CHUNK, zpad, 0)
        for cp in cps:
            cp.wait()

        def elem(e, carry):
            obase = e * SCORE_W
            vr = [v_rows[e, pl.ds(16 * j, 16)] for j in range(4)]
            up = [up_rows[e, pl.ds(16 * j, 16)] for j in range(4)]
            acc = vr[0] * up[0]
            for j in range(1, 4):
                acc = acc + vr[j] * up[j]
            s = _hsum(acc, perms)
            plsc.store_scatter(scores, [jnp.broadcast_to(obase, (16,))], s,
                               mask=lane0)
            nv = [-x for x in vr]
            for n in range(NEG):
                r = e * NEG + n
                un = [un_rows[r, pl.ds(16 * j, 16)] for j in range(4)]
                a = nv[0] * un[0]
                for j in range(1, 4):
                    a = a + nv[j] * un[j]
                s = _hsum(a, perms)
                plsc.store_scatter(scores,
                                   [jnp.broadcast_to(obase + n + 1, (16,))], s,
                                   mask=lane0)
            return carry

        lax.fori_loop(0, CHUNK, elem, 0)
        pltpu.sync_copy(scores,
                        out.at[pl.ds((base + cb) * SCORE_W, CHUNK * SCORE_W)])


def _tc_reduce(scores):
    nblk = 16

    def body(s_ref, o_ref):
        i = pl.program_id(0)

        @pl.when(i == 0)
        def _init():
            o_ref[0, 0] = jnp.float32(0.0)

        x = s_ref[...]
        col = lax.broadcasted_iota(jnp.int32, x.shape, 1)
        ls = jnp.minimum(x, 0.0) - jnp.log1p(jnp.exp(-jnp.abs(x)))
        ls = jnp.where(col < NEG + 1, ls, 0.0)
        o_ref[0, 0] += jnp.sum(ls)

    return pl.pallas_call(
        body,
        grid=(nblk,),
        in_specs=[pl.BlockSpec((B // nblk, SCORE_W), lambda i: (i, 0))],
        out_specs=pl.BlockSpec(memory_space=pltpu.SMEM),
        out_shape=jax.ShapeDtypeStruct((1, 1), jnp.float32),
    )(scores)


def kernel(v_emb, u_emb, v, u_pos, u_neg):
    v = v.astype(jnp.int32)
    u_pos = u_pos.astype(jnp.int32)
    u_neg_flat = u_neg.astype(jnp.int32).reshape(B * NEG)
    v_tail = lax.slice(v_emb, (N_FULL * CONV_W, 0), (1000000, D)).reshape(-1)
    u_tail = lax.slice(u_emb, (N_FULL * CONV_W, 0), (1000000, D)).reshape(-1)
    v_lin, u_lin = _sc_convert(v_emb.T, u_emb.T, v_tail, u_tail)
    scores = _sc_scores(v_lin.reshape(1000000, D), u_lin.reshape(1000000, D),
                        v, u_pos, u_neg_flat)
    tot = _tc_reduce(scores.reshape(B, SCORE_W))
    return -tot[0, 0]
